# baseline (speedup 1.0000x reference)
import jax
import jax.numpy as jnp
from jax.experimental import pallas as pl
from jax.experimental.pallas import tpu as pltpu

B, D, L = 64, 576, 196
Cc, K = 10, 20
M = Cc * K
LB = 49
NSTEPS = L // LB
CPAD = 128


def _partial_kernel(lit_ref, maskT_ref, out_ref):
    x = lit_ref[...].reshape(LB * B, D).astype(jnp.bfloat16)
    s = jnp.dot(x, maskT_ref[...].astype(jnp.bfloat16),
                preferred_element_type=jnp.float32)
    out_ref[...] = jnp.max(s.reshape(LB, B, M), axis=0)[None]


def _finish_kernel(part_ref, maskT_ref, alpha_ref, vote_ref, out_ref):
    smax = jnp.max(part_ref[...], axis=0)
    count = jnp.sum(maskT_ref[...], axis=0)
    clause = (smax >= count[None, :] - 0.5).astype(jnp.float32)
    weighted = clause * alpha_ref[...]
    out_ref[...] = jnp.dot(weighted, vote_ref[...],
                           preferred_element_type=jnp.float32)


def kernel(literals, clause_mask, alpha):
    lit_t = literals.transpose(2, 0, 1)
    maskT = clause_mask.T.astype(jnp.float32)
    alpha2 = alpha.reshape(1, M).astype(jnp.float32)
    m_idx = jnp.arange(M)
    cls = m_idx // K
    sign = jnp.where((m_idx % K) < (K // 2), 1.0, -1.0)
    vote = (sign[:, None] *
            (cls[:, None] == jnp.arange(CPAD)[None, :])).astype(jnp.float32)

    partial = pl.pallas_call(
        _partial_kernel,
        grid=(NSTEPS,),
        in_specs=[
            pl.BlockSpec((LB, B, D), lambda i: (i, 0, 0)),
            pl.BlockSpec((D, M), lambda i: (0, 0)),
        ],
        out_specs=pl.BlockSpec((1, B, M), lambda i: (i, 0, 0)),
        out_shape=jax.ShapeDtypeStruct((NSTEPS, B, M), jnp.float32),
        compiler_params=pltpu.CompilerParams(
            dimension_semantics=("parallel",),
        ),
    )(lit_t, maskT)

    out = pl.pallas_call(
        _finish_kernel,
        in_specs=[
            pl.BlockSpec((NSTEPS, B, M), lambda: (0, 0, 0)),
            pl.BlockSpec((D, M), lambda: (0, 0)),
            pl.BlockSpec((1, M), lambda: (0, 0)),
            pl.BlockSpec((M, CPAD), lambda: (0, 0)),
        ],
        out_specs=pl.BlockSpec((B, CPAD), lambda: (0, 0)),
        out_shape=jax.ShapeDtypeStruct((B, CPAD), jnp.float32),
    )(partial, maskT, alpha2, vote)
    return out[:, :Cc]


# speedup vs baseline: 1.1043x; 1.1043x over previous
import jax
import jax.numpy as jnp
from jax.experimental import pallas as pl
from jax.experimental.pallas import tpu as pltpu

B, D, L = 64, 576, 196
Cc, K = 10, 20
M = Cc * K
LB = 49
NSTEPS = L // LB
CPAD = 128


def _tm_kernel(lit_ref, maskT_ref, alpha_ref, vote_ref, out_ref, acc_ref):
    i = pl.program_id(0)
    maskT = maskT_ref[...]
    x = lit_ref[...].reshape(LB * B, D).astype(jnp.bfloat16)
    s = jnp.dot(x, maskT.astype(jnp.bfloat16),
                preferred_element_type=jnp.float32)
    m = jnp.max(s.reshape(LB, B, M), axis=0)

    @pl.when(i == 0)
    def _init():
        acc_ref[...] = m

    @pl.when(i > 0)
    def _acc():
        acc_ref[...] = jnp.maximum(acc_ref[...], m)

    @pl.when(i == NSTEPS - 1)
    def _finish():
        count = jnp.sum(maskT, axis=0)
        clause = (acc_ref[...] >= count[None, :] - 0.5).astype(jnp.float32)
        weighted = clause * alpha_ref[...]
        out_ref[...] = jnp.dot(weighted, vote_ref[...],
                               preferred_element_type=jnp.float32)


def kernel(literals, clause_mask, alpha):
    lit_t = literals.transpose(2, 0, 1)
    maskT = clause_mask.T.astype(jnp.float32)
    alpha2 = alpha.reshape(1, M).astype(jnp.float32)
    m_idx = jnp.arange(M)
    cls = m_idx // K
    sign = jnp.where((m_idx % K) < (K // 2), 1.0, -1.0)
    vote = (sign[:, None] *
            (cls[:, None] == jnp.arange(CPAD)[None, :])).astype(jnp.float32)

    out = pl.pallas_call(
        _tm_kernel,
        grid=(NSTEPS,),
        in_specs=[
            pl.BlockSpec((LB, B, D), lambda i: (i, 0, 0)),
            pl.BlockSpec((D, M), lambda i: (0, 0)),
            pl.BlockSpec((1, M), lambda i: (0, 0)),
            pl.BlockSpec((M, CPAD), lambda i: (0, 0)),
        ],
        out_specs=pl.BlockSpec((B, CPAD), lambda i: (0, 0)),
        out_shape=jax.ShapeDtypeStruct((B, CPAD), jnp.float32),
        scratch_shapes=[pltpu.VMEM((B, M), jnp.float32)],
        compiler_params=pltpu.CompilerParams(
            dimension_semantics=("arbitrary",),
        ),
    )(lit_t, maskT, alpha2, vote)
    return out[:, :Cc]


# speedup vs baseline: 1.2230x; 1.1075x over previous
import jax
import jax.numpy as jnp
from jax.experimental import pallas as pl
from jax.experimental.pallas import tpu as pltpu

B, D, L = 64, 576, 196
Cc, K = 10, 20
M = Cc * K
LB = 49
NSTEPS = L // LB
CPAD = 128


def _tm_kernel(lit_ref, maskT_ref, alpha_ref, vote_ref, out_ref, acc_ref):
    i = pl.program_id(0)
    maskT = maskT_ref[...]
    x = lit_ref[...].reshape(LB * B, D).astype(jnp.bfloat16)
    s = jnp.dot(x, maskT.astype(jnp.bfloat16),
                preferred_element_type=jnp.float32)
    m = jnp.max(s.reshape(LB, B, M), axis=0)

    @pl.when(i == 0)
    def _init():
        acc_ref[...] = m

    @pl.when(i > 0)
    def _acc():
        acc_ref[...] = jnp.maximum(acc_ref[...], m)

    @pl.when(i == NSTEPS - 1)
    def _finish():
        count = jnp.sum(maskT, axis=0)
        clause = (acc_ref[...] >= count[None, :] - 0.5).astype(jnp.float32)
        weighted = clause * alpha_ref[...]
        out_ref[...] = jax.lax.dot_general(
            vote_ref[...], weighted, (((1,), (1,)), ((), ())),
            preferred_element_type=jnp.float32)


def kernel(literals, clause_mask, alpha):
    lit_t = literals.transpose(2, 0, 1)
    maskT = clause_mask.T.astype(jnp.float32)
    alpha2 = alpha.reshape(1, M).astype(jnp.float32)
    m_idx = jnp.arange(M)
    cls = m_idx // K
    sign = jnp.where((m_idx % K) < (K // 2), 1.0, -1.0)
    voteT = (sign[None, :] *
             (cls[None, :] == jnp.arange(Cc)[:, None])).astype(jnp.float32)

    out = pl.pallas_call(
        _tm_kernel,
        grid=(NSTEPS,),
        in_specs=[
            pl.BlockSpec((LB, B, D), lambda i: (i, 0, 0)),
            pl.BlockSpec((D, M), lambda i: (0, 0)),
            pl.BlockSpec((1, M), lambda i: (0, 0)),
            pl.BlockSpec((Cc, M), lambda i: (0, 0)),
        ],
        out_specs=pl.BlockSpec((Cc, B), lambda i: (0, 0)),
        out_shape=jax.ShapeDtypeStruct((Cc, B), jnp.float32),
        scratch_shapes=[pltpu.VMEM((B, M), jnp.float32)],
        compiler_params=pltpu.CompilerParams(
            dimension_semantics=("arbitrary",),
        ),
    )(lit_t, maskT, alpha2, voteT)
    return out.T


# speedup vs baseline: 1.2753x; 1.0427x over previous
import jax
import jax.numpy as jnp
from jax.experimental import pallas as pl
from jax.experimental.pallas import tpu as pltpu

B, D, L = 64, 576, 196
Cc, K = 10, 20
M = Cc * K
LB = 49
NSTEPS = L // LB
CPAD = 128


def _tm_kernel(lit_ref, maskT_ref, alpha_ref, out_ref, acc_ref):
    i = pl.program_id(0)
    maskT = maskT_ref[...]
    x = lit_ref[...].reshape(LB * B, D).astype(jnp.bfloat16)
    s = jnp.dot(x, maskT.astype(jnp.bfloat16),
                preferred_element_type=jnp.float32)
    m = jnp.max(s.reshape(LB, B, M), axis=0)

    @pl.when(i == 0)
    def _init():
        acc_ref[...] = m

    @pl.when(i > 0)
    def _acc():
        acc_ref[...] = jnp.maximum(acc_ref[...], m)

    @pl.when(i == NSTEPS - 1)
    def _finish():
        count = jnp.sum(maskT, axis=0)
        clause = (acc_ref[...] >= count[None, :] - 0.5).astype(jnp.float32)
        weighted = clause * alpha_ref[...]
        m_idx = jax.lax.broadcasted_iota(jnp.int32, (Cc, M), 1)
        c_idx = jax.lax.broadcasted_iota(jnp.int32, (Cc, M), 0)
        sign = jnp.where((m_idx % K) < (K // 2), 1.0, -1.0)
        voteT = jnp.where(m_idx // K == c_idx, sign, 0.0)
        out_ref[...] = jax.lax.dot_general(
            voteT, weighted, (((1,), (1,)), ((), ())),
            preferred_element_type=jnp.float32)


def kernel(literals, clause_mask, alpha):
    lit_t = literals.transpose(2, 0, 1)
    maskT = clause_mask.T.astype(jnp.float32)
    alpha2 = alpha.reshape(1, M).astype(jnp.float32)

    out = pl.pallas_call(
        _tm_kernel,
        grid=(NSTEPS,),
        in_specs=[
            pl.BlockSpec((LB, B, D), lambda i: (i, 0, 0)),
            pl.BlockSpec((D, M), lambda i: (0, 0)),
            pl.BlockSpec((1, M), lambda i: (0, 0)),
        ],
        out_specs=pl.BlockSpec((Cc, B), lambda i: (0, 0)),
        out_shape=jax.ShapeDtypeStruct((Cc, B), jnp.float32),
        scratch_shapes=[pltpu.VMEM((B, M), jnp.float32)],
        compiler_params=pltpu.CompilerParams(
            dimension_semantics=("arbitrary",),
        ),
    )(lit_t, maskT, alpha2)
    return out.T


# speedup vs baseline: 1.2919x; 1.0130x over previous
import jax
import jax.numpy as jnp
from jax.experimental import pallas as pl
from jax.experimental.pallas import tpu as pltpu

B, D, L = 64, 576, 196
Cc, K = 10, 20
M = Cc * K
LB = 49
NSTEPS = L // LB
CPAD = 128


def _tm_kernel(lit_ref, mask_ref, alpha_ref, out_ref, acc_ref):
    i = pl.program_id(0)
    mask = mask_ref[...]
    x = lit_ref[...].reshape(LB * B, D).astype(jnp.bfloat16)
    s = jax.lax.dot_general(x, mask.astype(jnp.bfloat16),
                            (((1,), (1,)), ((), ())),
                            preferred_element_type=jnp.float32)
    m = jnp.max(s.reshape(LB, B, M), axis=0)

    @pl.when(i == 0)
    def _init():
        acc_ref[...] = m

    @pl.when(i > 0)
    def _acc():
        acc_ref[...] = jnp.maximum(acc_ref[...], m)

    @pl.when(i == NSTEPS - 1)
    def _finish():
        count = jnp.sum(mask, axis=1)
        clause = (acc_ref[...] >= count[None, :] - 0.5).astype(jnp.float32)
        weighted = clause * alpha_ref[...]
        m_idx = jax.lax.broadcasted_iota(jnp.int32, (Cc, M), 1)
        c_idx = jax.lax.broadcasted_iota(jnp.int32, (Cc, M), 0)
        sign = jnp.where((m_idx % K) < (K // 2), 1.0, -1.0)
        voteT = jnp.where(m_idx // K == c_idx, sign, 0.0)
        out_ref[...] = jax.lax.dot_general(
            voteT, weighted, (((1,), (1,)), ((), ())),
            preferred_element_type=jnp.float32)


def kernel(literals, clause_mask, alpha):
    lit_t = literals.transpose(2, 0, 1)
    mask_f = clause_mask.astype(jnp.float32)
    alpha2 = alpha.reshape(1, M).astype(jnp.float32)

    out = pl.pallas_call(
        _tm_kernel,
        grid=(NSTEPS,),
        in_specs=[
            pl.BlockSpec((LB, B, D), lambda i: (i, 0, 0)),
            pl.BlockSpec((M, D), lambda i: (0, 0)),
            pl.BlockSpec((1, M), lambda i: (0, 0)),
        ],
        out_specs=pl.BlockSpec((Cc, B), lambda i: (0, 0)),
        out_shape=jax.ShapeDtypeStruct((Cc, B), jnp.float32),
        scratch_shapes=[pltpu.VMEM((B, M), jnp.float32)],
        compiler_params=pltpu.CompilerParams(
            dimension_semantics=("arbitrary",),
        ),
    )(lit_t, mask_f, alpha2)
    return out.T
